# baseline (speedup 1.0000x reference)
import functools

import jax
import jax.numpy as jnp
from jax import lax
from jax.experimental import pallas as pl
from jax.experimental.pallas import tpu as pltpu
from jax.experimental.pallas import tpu_sc as plsc

PAD_IDX = 1
LN_EPS = 1e-05
LANES = 16
CHUNK = 8
NSETS = 4


def _prep_body(ids_ref, spk_ref, pos_ref, type_ref, ids3_ref, posplus_ref):
    ids = ids_ref[...]
    mask = (ids != PAD_IDX).astype(jnp.int32)
    s = ids.shape[1]
    inc = mask
    d = 1
    while d < s:
        shifted = jnp.concatenate(
            [jnp.zeros((ids.shape[0], d), jnp.int32), inc[:, :-d]], axis=1)
        inc = inc + shifted
        d *= 2
    ids3_ref[0] = ids
    ids3_ref[1] = spk_ref[...]
    ids3_ref[2] = inc * mask + PAD_IDX
    posplus_ref[...] = pos_ref[...] + type_ref[0, :][None, :]


def _lane_sum(x):
    lane = lax.iota(jnp.int32, LANES)
    for d in (8, 4, 2, 1):
        x = x + x.at[lane ^ d].get(mode="promise_in_bounds")
    return x


def _make_sc_kernel(n_tokens, hidden, n_workers):
    tpw = n_tokens // n_workers
    n_chunks = tpw // CHUNK
    n_slices = hidden // LANES
    mesh = plsc.VectorSubcoreMesh(core_axis_name="c", subcore_axis_name="s")

    @functools.partial(
        pl.kernel,
        mesh=mesh,
        out_type=jax.ShapeDtypeStruct((n_tokens, hidden), jnp.float32),
        scratch_types=(
            [pltpu.VMEM((3, tpw), jnp.int32),
             pltpu.VMEM((3, CHUNK), jnp.int32)]
            + [pltpu.VMEM((CHUNK, hidden), jnp.float32)
               for _ in range(4 * NSETS)]
            + [pltpu.VMEM((hidden,), jnp.float32),
               pltpu.VMEM((hidden,), jnp.float32),
               pltpu.SMEM((2 * CHUNK,), jnp.float32)]
            + [pltpu.SemaphoreType.DMA
               for _ in range(2 * NSETS)]
        ),
    )
    def sc_kernel(ids_hbm,
                  word_hbm, pos_hbm, spk_hbm, gamma_hbm, beta_hbm,
                  out_hbm,
                  vi, vi16, *rest):
        row_bufs = rest[:4 * NSETS]
        g_v, b_v, stats = rest[4 * NSETS:4 * NSETS + 3]
        sems = rest[4 * NSETS + 3:]
        bufs = tuple(
            (row_bufs[4 * i], row_bufs[4 * i + 1], row_bufs[4 * i + 2],
             row_bufs[4 * i + 3], sems[2 * i], sems[2 * i + 1])
            for i in range(NSETS))
        wid = lax.axis_index("s") * 2 + lax.axis_index("c")
        base = wid * tpw

        pltpu.sync_copy(ids_hbm.at[0, pl.ds(base, CHUNK)], vi16.at[0])
        pltpu.sync_copy(ids_hbm.at[1, pl.ds(base, CHUNK)], vi16.at[1])
        pltpu.sync_copy(ids_hbm.at[2, pl.ds(base, CHUNK)], vi16.at[2])
        pltpu.async_copy(word_hbm.at[vi16.at[0]], bufs[0][0], bufs[0][4])
        pltpu.async_copy(spk_hbm.at[vi16.at[1]], bufs[0][1], bufs[0][4])
        pltpu.async_copy(pos_hbm.at[vi16.at[2]], bufs[0][2], bufs[0][4])
        pltpu.sync_copy(ids_hbm.at[:, pl.ds(base, tpw)], vi)
        pltpu.sync_copy(gamma_hbm, g_v)
        pltpu.sync_copy(beta_hbm, b_v)

        def issue_gathers(c, parity):
            bw, bs, bp, _, sem, _ = bufs[parity]
            off = c * CHUNK
            pltpu.async_copy(word_hbm.at[vi.at[0, pl.ds(off, CHUNK)]], bw, sem)
            pltpu.async_copy(spk_hbm.at[vi.at[1, pl.ds(off, CHUNK)]], bs, sem)
            pltpu.async_copy(pos_hbm.at[vi.at[2, pl.ds(off, CHUNK)]], bp, sem)

        def wait_gathers(parity):
            bw, bs, bp, _, sem, _ = bufs[parity]
            zidx = vi.at[2, pl.ds(0, CHUNK)]
            pltpu.make_async_copy(pos_hbm.at[zidx], bw, sem).wait()
            pltpu.make_async_copy(pos_hbm.at[zidx], bs, sem).wait()
            pltpu.make_async_copy(pos_hbm.at[zidx], bp, sem).wait()

        def wait_out(parity):
            _, _, _, o, _, sem = bufs[parity]
            pltpu.make_async_copy(o, out_hbm.at[pl.ds(base, CHUNK)], sem).wait()

        def compute_chunk(c, parity):
            bw, bs, bp, o, _, sem_o = bufs[parity]
            off = base + c * CHUNK

            def p1_body(j, accs):
                acc_s, acc_q = accs
                sl = pl.ds(j * LANES, LANES)
                new_s = []
                new_q = []
                for t in range(CHUNK):
                    x = bw[t, sl] + bs[t, sl] + bp[t, sl]
                    o[t, sl] = x
                    new_s.append(acc_s[t] + x)
                    new_q.append(acc_q[t] + x * x)
                return tuple(new_s), tuple(new_q)

            zeros = tuple(jnp.zeros((LANES,), jnp.float32)
                          for _ in range(CHUNK))
            acc_s, acc_q = lax.fori_loop(0, n_slices, p1_body, (zeros, zeros))

            for t in range(CHUNK):
                ssum = _lane_sum(acc_s[t])[0]
                qsum = _lane_sum(acc_q[t])[0]
                mean = ssum * (1.0 / hidden)
                var = qsum * (1.0 / hidden) - mean * mean
                v = var + LN_EPS
                vi_ = lax.bitcast_convert_type(v, jnp.int32)
                y = lax.bitcast_convert_type(
                    jnp.int32(0x5F3759DF) - (vi_ >> 1), jnp.float32)
                hv = 0.5 * v
                y = y * (1.5 - hv * y * y)
                y = y * (1.5 - hv * y * y)
                y = y * (1.5 - hv * y * y)
                stats[t] = mean
                stats[t + CHUNK] = y

            def p2_body(j, carry_j):
                sl = pl.ds(j * LANES, LANES)
                g = g_v[sl]
                b = b_v[sl]
                for t in range(CHUNK):
                    x = o[t, sl]
                    o[t, sl] = (x - stats[t]) * stats[t + CHUNK] * g + b
                return carry_j

            lax.fori_loop(0, n_slices, p2_body, 0)
            pltpu.async_copy(o, out_hbm.at[pl.ds(off, CHUNK)], sem_o)

        for c0 in range(1, NSETS - 1):
            issue_gathers(c0, c0)

        def round_body(k, carry_k):
            for i in range(NSETS):
                c = NSETS * k + i

                @pl.when(c + NSETS - 1 < n_chunks)
                def _():
                    issue_gathers(c + NSETS - 1, (i + NSETS - 1) % NSETS)

                wait_gathers(i)

                @pl.when(k > 0)
                def _():
                    wait_out(i)

                compute_chunk(c, i)
            return carry_k

        lax.fori_loop(0, n_chunks // NSETS, round_body, 0)
        for i in range(NSETS):
            wait_out(i)

    return sc_kernel


def kernel(input_ids, speaker_ids, word_table, pos_table, type_table,
           speaker_table, ln_gamma, ln_beta):
    b, s = input_ids.shape
    hidden = word_table.shape[1]
    max_pos = pos_table.shape[0]
    n = b * s

    ids3, pos_plus = pl.pallas_call(
        _prep_body,
        out_shape=(
            jax.ShapeDtypeStruct((3, b, s), jnp.int32),
            jax.ShapeDtypeStruct((max_pos, hidden), jnp.float32),
        ),
    )(input_ids.astype(jnp.int32), speaker_ids.astype(jnp.int32),
      pos_table, type_table)

    sc = _make_sc_kernel(n, hidden, n_workers=32)
    out = sc(
        ids3.reshape(3, n),
        word_table, pos_plus, speaker_table,
        ln_gamma, ln_beta,
    )
    return out.reshape(b, s, hidden)


# speedup vs baseline: 1.1281x; 1.1281x over previous
import functools

import jax
import jax.numpy as jnp
from jax import lax
from jax.experimental import pallas as pl
from jax.experimental.pallas import tpu as pltpu
from jax.experimental.pallas import tpu_sc as plsc

PAD_IDX = 1
LN_EPS = 1e-05
LANES = 16
CHUNK = 16


def _prep_body(ids_ref, spk_ref, pos_ref, type_ref, ids3_ref, posplus_ref):
    ids = ids_ref[...]
    mask = (ids != PAD_IDX).astype(jnp.int32)
    s = ids.shape[1]
    inc = mask
    d = 1
    while d < s:
        shifted = jnp.concatenate(
            [jnp.zeros((ids.shape[0], d), jnp.int32), inc[:, :-d]], axis=1)
        inc = inc + shifted
        d *= 2
    ids3_ref[0] = ids
    ids3_ref[1] = spk_ref[...]
    ids3_ref[2] = inc * mask + PAD_IDX
    posplus_ref[...] = pos_ref[...] + type_ref[0, :][None, :]


def _lane_sum(x):
    lane = lax.iota(jnp.int32, LANES)
    for d in (8, 4, 2, 1):
        x = x + x.at[lane ^ d].get(mode="promise_in_bounds")
    return x


def _make_sc_kernel(n_tokens, hidden, n_workers):
    tpw = n_tokens // n_workers
    n_chunks = tpw // CHUNK
    n_slices = hidden // LANES
    mesh = plsc.VectorSubcoreMesh(core_axis_name="c", subcore_axis_name="s")

    @functools.partial(
        pl.kernel,
        mesh=mesh,
        out_type=jax.ShapeDtypeStruct((n_tokens, hidden), jnp.float32),
        scratch_types=[
            pltpu.VMEM((3, tpw), jnp.int32),
            pltpu.VMEM((3, CHUNK), jnp.int32),
            pltpu.VMEM((CHUNK, hidden), jnp.float32),
            pltpu.VMEM((CHUNK, hidden), jnp.float32),
            pltpu.VMEM((CHUNK, hidden), jnp.float32),
            pltpu.VMEM((CHUNK, hidden), jnp.float32),
            pltpu.VMEM((CHUNK, hidden), jnp.float32),
            pltpu.VMEM((CHUNK, hidden), jnp.float32),
            pltpu.VMEM((CHUNK, hidden), jnp.float32),
            pltpu.VMEM((CHUNK, hidden), jnp.float32),
            pltpu.VMEM((hidden,), jnp.float32),
            pltpu.VMEM((hidden,), jnp.float32),
            pltpu.SMEM((2 * LANES,), jnp.float32),
            pltpu.SemaphoreType.DMA,
            pltpu.SemaphoreType.DMA,
            pltpu.SemaphoreType.DMA,
            pltpu.SemaphoreType.DMA,
        ],
    )
    def sc_kernel(ids_hbm,
                  word_hbm, pos_hbm, spk_hbm, gamma_hbm, beta_hbm,
                  out_hbm,
                  vi, vi16, bw0, bw1, bs0, bs1, bp0, bp1, o0, o1,
                  g_v, b_v, stats, sem_g0, sem_g1, sem_o0, sem_o1):
        wid = lax.axis_index("s") * 2 + lax.axis_index("c")
        base = wid * tpw
        bufs = ((bw0, bs0, bp0, o0, sem_g0, sem_o0),
                (bw1, bs1, bp1, o1, sem_g1, sem_o1))

        pltpu.sync_copy(ids_hbm.at[0, pl.ds(base, CHUNK)], vi16.at[0])
        pltpu.sync_copy(ids_hbm.at[1, pl.ds(base, CHUNK)], vi16.at[1])
        pltpu.sync_copy(ids_hbm.at[2, pl.ds(base, CHUNK)], vi16.at[2])
        pltpu.async_copy(word_hbm.at[vi16[0]], bw0, sem_g0)
        pltpu.async_copy(spk_hbm.at[vi16[1]], bs0, sem_g0)
        pltpu.async_copy(pos_hbm.at[vi16[2]], bp0, sem_g0)
        pltpu.sync_copy(ids_hbm.at[:, pl.ds(base, tpw)], vi)
        pltpu.sync_copy(gamma_hbm, g_v)
        pltpu.sync_copy(beta_hbm, b_v)

        def issue_gathers(c, parity):
            bw, bs, bp, _, sem, _ = bufs[parity]
            off = c * CHUNK
            pltpu.async_copy(word_hbm.at[vi[0, pl.ds(off, CHUNK)]], bw, sem)
            pltpu.async_copy(spk_hbm.at[vi[1, pl.ds(off, CHUNK)]], bs, sem)
            pltpu.async_copy(pos_hbm.at[vi[2, pl.ds(off, CHUNK)]], bp, sem)

        def wait_gathers(parity):
            bw, bs, bp, _, sem, _ = bufs[parity]
            zidx = vi[2, pl.ds(0, CHUNK)]
            pltpu.make_async_copy(pos_hbm.at[zidx], bw, sem).wait()
            pltpu.make_async_copy(pos_hbm.at[zidx], bs, sem).wait()
            pltpu.make_async_copy(pos_hbm.at[zidx], bp, sem).wait()

        def wait_out(parity):
            _, _, _, o, _, sem = bufs[parity]
            pltpu.make_async_copy(o, out_hbm.at[pl.ds(base, CHUNK)], sem).wait()

        def compute_chunk(c, parity):
            bw, bs, bp, o, _, sem_o = bufs[parity]
            off = base + c * CHUNK

            def p1_body(j, accs):
                acc_s, acc_q = accs
                sl = pl.ds(j * LANES, LANES)
                new_s = []
                new_q = []
                for t in range(CHUNK):
                    x = bw[t, sl] + bs[t, sl] + bp[t, sl]
                    o[t, sl] = x
                    new_s.append(acc_s[t] + x)
                    new_q.append(acc_q[t] + x * x)
                return tuple(new_s), tuple(new_q)

            zeros = tuple(jnp.zeros((LANES,), jnp.float32)
                          for _ in range(CHUNK))
            acc_s, acc_q = lax.fori_loop(0, n_slices, p1_body, (zeros, zeros))

            for t in range(CHUNK):
                ssum = _lane_sum(acc_s[t])[0]
                qsum = _lane_sum(acc_q[t])[0]
                mean = ssum * (1.0 / hidden)
                var = qsum * (1.0 / hidden) - mean * mean
                v = var + LN_EPS
                vi_ = lax.bitcast_convert_type(v, jnp.int32)
                y = lax.bitcast_convert_type(
                    jnp.int32(0x5F3759DF) - (vi_ >> 1), jnp.float32)
                hv = 0.5 * v
                y = y * (1.5 - hv * y * y)
                y = y * (1.5 - hv * y * y)
                y = y * (1.5 - hv * y * y)
                stats[t] = mean
                stats[t + LANES] = y

            def p2_body(j, carry_j):
                sl = pl.ds(j * LANES, LANES)
                g = g_v[sl]
                b = b_v[sl]
                for t in range(CHUNK):
                    x = o[t, sl]
                    o[t, sl] = (x - stats[t]) * stats[t + LANES] * g + b
                return carry_j

            lax.fori_loop(0, n_slices, p2_body, 0)
            pltpu.async_copy(o, out_hbm.at[pl.ds(off, CHUNK)], sem_o)

        def pair_body(k, carry_k):
            cA = 2 * k
            issue_gathers(cA + 1, 1)
            wait_gathers(0)

            @pl.when(k > 0)
            def _():
                wait_out(0)

            compute_chunk(cA, 0)

            @pl.when(k < n_chunks // 2 - 1)
            def _():
                issue_gathers(cA + 2, 0)

            wait_gathers(1)

            @pl.when(k > 0)
            def _():
                wait_out(1)

            compute_chunk(cA + 1, 1)
            return carry_k

        lax.fori_loop(0, n_chunks // 2, pair_body, 0)
        wait_out(0)
        wait_out(1)

    return sc_kernel


def kernel(input_ids, speaker_ids, word_table, pos_table, type_table,
           speaker_table, ln_gamma, ln_beta):
    b, s = input_ids.shape
    hidden = word_table.shape[1]
    max_pos = pos_table.shape[0]
    n = b * s

    ids3, pos_plus = pl.pallas_call(
        _prep_body,
        out_shape=(
            jax.ShapeDtypeStruct((3, b, s), jnp.int32),
            jax.ShapeDtypeStruct((max_pos, hidden), jnp.float32),
        ),
    )(input_ids.astype(jnp.int32), speaker_ids.astype(jnp.int32),
      pos_table, type_table)

    sc = _make_sc_kernel(n, hidden, n_workers=32)
    out = sc(
        ids3.reshape(3, n),
        word_table, pos_plus, speaker_table,
        ln_gamma, ln_beta,
    )
    return out.reshape(b, s, hidden)
